# baseline (speedup 1.0000x reference)
import jax
import jax.numpy as jnp
import numpy as np
from jax.experimental import pallas as pl

def _copy_body(x_ref, o_ref):
    o_ref[...] = x_ref[...]


def kernel(x):
    b, c, h, w = x.shape
    npairs = c // 2
    x2 = x.reshape(b, npairs, 2, h * w)
    out2 = pl.pallas_call(
        _copy_body,
        grid=(npairs,),
        in_specs=[
            pl.BlockSpec((b, 1, 2, h * w), lambda p: (0, (35 * p) % npairs, 0, 0)),
        ],
        out_specs=pl.BlockSpec((b, 1, 2, h * w), lambda p: (0, p, 0, 0)),
        out_shape=jax.ShapeDtypeStruct((b, npairs, 2, h * w), x.dtype),
    )(x2)
    return out2.reshape(b, c, h, w)
